# baseline (speedup 1.0000x reference)
import jax
import jax.numpy as jnp
from jax.experimental import pallas as pl
from jax.experimental.pallas import tpu as pltpu

_PARALLEL = pltpu.CompilerParams(dimension_semantics=("parallel",))

N = 3327
IN_F = 3703
OUT_F = 64
ALPHA = 0.2
BR = 128

_DEF = jax.lax.Precision.DEFAULT


def _proj_body(h_ref, w_ref, a_ref, wh_ref, wh1_ref, wh2_ref):
    wh = jnp.dot(h_ref[...], w_ref[...], precision=_DEF,
                 preferred_element_type=jnp.float32)
    wh_ref[...] = wh
    a1 = a_ref[:OUT_F, :]
    a2 = a_ref[OUT_F:, :]
    wh1_ref[...] = jnp.dot(wh, a1, precision=_DEF,
                           preferred_element_type=jnp.float32)
    wh2_ref[...] = jax.lax.dot_general(
        a2, wh, dimension_numbers=(((0,), (1,)), ((), ())), precision=_DEF,
        preferred_element_type=jnp.float32)


def _attn_body(adj_ref, wh1_ref, wh2_ref, whf_ref, out_ref):
    logits = wh1_ref[...] + wh2_ref[...]
    e = jnp.maximum(logits, ALPHA * logits)
    masked = jnp.where(adj_ref[...] > 0, e, jnp.float32(-9e15))
    m = jnp.max(masked, axis=1, keepdims=True)
    p = jnp.exp(masked - m)
    s = jnp.sum(p, axis=1, keepdims=True)
    hp = jnp.dot(p, whf_ref[...], precision=_DEF,
                 preferred_element_type=jnp.float32)
    hp = hp / s
    out_ref[...] = jnp.where(hp > 0, hp, jnp.exp(hp) - 1.0)


@jax.jit
def kernel(h, adj, W, a):
    grid = (pl.cdiv(N, BR),)
    wh, wh1, wh2 = pl.pallas_call(
        _proj_body,
        grid=grid,
        in_specs=[
            pl.BlockSpec((BR, IN_F), lambda i: (i, 0)),
            pl.BlockSpec((IN_F, OUT_F), lambda i: (0, 0)),
            pl.BlockSpec((2 * OUT_F, 1), lambda i: (0, 0)),
        ],
        out_specs=[
            pl.BlockSpec((BR, OUT_F), lambda i: (i, 0)),
            pl.BlockSpec((BR, 1), lambda i: (i, 0)),
            pl.BlockSpec((1, BR), lambda i: (0, i)),
        ],
        out_shape=[
            jax.ShapeDtypeStruct((N, OUT_F), jnp.float32),
            jax.ShapeDtypeStruct((N, 1), jnp.float32),
            jax.ShapeDtypeStruct((1, N), jnp.float32),
        ],
        compiler_params=_PARALLEL,
    )(h, W, a)

    out = pl.pallas_call(
        _attn_body,
        grid=grid,
        in_specs=[
            pl.BlockSpec((BR, N), lambda i: (i, 0)),
            pl.BlockSpec((BR, 1), lambda i: (i, 0)),
            pl.BlockSpec((1, N), lambda i: (0, 0)),
            pl.BlockSpec((N, OUT_F), lambda i: (0, 0)),
        ],
        out_specs=pl.BlockSpec((BR, OUT_F), lambda i: (i, 0)),
        out_shape=jax.ShapeDtypeStruct((N, OUT_F), jnp.float32),
        compiler_params=_PARALLEL,
    )(adj, wh1, wh2, wh)
    return out


# speedup vs baseline: 1.1395x; 1.1395x over previous
import jax
import jax.numpy as jnp
from jax.experimental import pallas as pl
from jax.experimental.pallas import tpu as pltpu

_PARALLEL = pltpu.CompilerParams(dimension_semantics=("parallel",))

N = 3327
IN_F = 3703
OUT_F = 64
ALPHA = 0.2
BR = 256

_DEF = jax.lax.Precision.DEFAULT


def _proj_body(h_ref, w_ref, a_ref, wh_ref, wh1_ref, wh2_ref):
    wh = jnp.dot(h_ref[...], w_ref[...], precision=_DEF,
                 preferred_element_type=jnp.float32)
    wh_ref[...] = wh
    a1 = a_ref[:OUT_F, :]
    a2 = a_ref[OUT_F:, :]
    wh1_ref[...] = jnp.dot(wh, a1, precision=_DEF,
                           preferred_element_type=jnp.float32)
    wh2_ref[...] = jax.lax.dot_general(
        a2, wh, dimension_numbers=(((0,), (1,)), ((), ())), precision=_DEF,
        preferred_element_type=jnp.float32)


def _attn_body(adj_ref, wh1_ref, wh2_ref, whf_ref, out_ref):
    logits = wh1_ref[...] + wh2_ref[...]
    e = jnp.maximum(logits, ALPHA * logits)
    masked = jnp.where(adj_ref[...] > 0, e, jnp.float32(-9e15))
    m = jnp.max(masked, axis=1, keepdims=True)
    p = jnp.exp(masked - m)
    s = jnp.sum(p, axis=1, keepdims=True)
    hp = jnp.dot(p, whf_ref[...], precision=_DEF,
                 preferred_element_type=jnp.float32)
    hp = hp / s
    out_ref[...] = jnp.where(hp > 0, hp, jnp.exp(hp) - 1.0)


@jax.jit
def kernel(h, adj, W, a):
    grid = (pl.cdiv(N, BR),)
    wh, wh1, wh2 = pl.pallas_call(
        _proj_body,
        grid=grid,
        in_specs=[
            pl.BlockSpec((BR, IN_F), lambda i: (i, 0)),
            pl.BlockSpec((IN_F, OUT_F), lambda i: (0, 0)),
            pl.BlockSpec((2 * OUT_F, 1), lambda i: (0, 0)),
        ],
        out_specs=[
            pl.BlockSpec((BR, OUT_F), lambda i: (i, 0)),
            pl.BlockSpec((BR, 1), lambda i: (i, 0)),
            pl.BlockSpec((1, BR), lambda i: (0, i)),
        ],
        out_shape=[
            jax.ShapeDtypeStruct((N, OUT_F), jnp.float32),
            jax.ShapeDtypeStruct((N, 1), jnp.float32),
            jax.ShapeDtypeStruct((1, N), jnp.float32),
        ],
        compiler_params=_PARALLEL,
    )(h, W, a)

    out = pl.pallas_call(
        _attn_body,
        grid=grid,
        in_specs=[
            pl.BlockSpec((BR, N), lambda i: (i, 0)),
            pl.BlockSpec((BR, 1), lambda i: (i, 0)),
            pl.BlockSpec((1, N), lambda i: (0, 0)),
            pl.BlockSpec((N, OUT_F), lambda i: (0, 0)),
        ],
        out_specs=pl.BlockSpec((BR, OUT_F), lambda i: (i, 0)),
        out_shape=jax.ShapeDtypeStruct((N, OUT_F), jnp.float32),
        compiler_params=_PARALLEL,
    )(adj, wh1, wh2, wh)
    return out
